# Initial kernel scaffold; baseline (speedup 1.0000x reference)
import jax
import jax.numpy as jnp
from jax.experimental import pallas as pl


def kernel(disp_preds, keysets, lambda_sets):
    raise NotImplementedError("write your pallas kernel here")


# speedup vs baseline: 2.6104x; 2.6104x over previous
import functools

import jax
import jax.numpy as jnp
from jax import lax
from jax.experimental import pallas as pl
from jax.experimental.pallas import tpu as pltpu
from jax.experimental.pallas import tpu_sc as plsc

_NC = 2
_NS = 16
_NW = _NC * _NS
_L = 16

_HW = 512 * 512
_K = 4096
_NIDX = 3 * _K
_CH = 128
_NCHUNK = _NIDX // _CH
_NBUF = 8


def _sc_fsloss(disp_flat, idx2d, lam2d):
    mesh = plsc.VectorSubcoreMesh(core_axis_name="c", subcore_axis_name="s")

    @functools.partial(
        pl.kernel,
        out_type=jax.ShapeDtypeStruct((_NW, _L), jnp.float32),
        mesh=mesh,
        scratch_types=[
            pltpu.VMEM((_NIDX,), jnp.int32),
            pltpu.VMEM((_NIDX,), jnp.float32),
            pltpu.VMEM((_K,), jnp.float32),
            pltpu.VMEM((_L,), jnp.float32),
            pltpu.SemaphoreType.DMA,
        ],
    )
    def run(disp_hbm, idx_hbm, lam_hbm, out_hbm, idx_v, vals_v, lam_v,
            part_v, sem):
        w = lax.axis_index("s") * _NC + lax.axis_index("c")
        b = lax.rem(w, 4)

        pltpu.sync_copy(idx_hbm.at[b], idx_v)
        pltpu.sync_copy(lam_hbm.at[b], lam_v)

        offv = jnp.full((_L,), w * _HW, dtype=jnp.int32)

        def add_off(k, carry):
            sl = pl.ds(k * _L, _L)
            idx_v[sl] = idx_v[sl] + offv
            return carry

        lax.fori_loop(0, _NIDX // _L, add_off, 0)

        def gather_group(g, carry):
            base = g * (_NBUF * _CH)
            copies = []
            for j in range(_NBUF):
                sl = pl.ds(base + j * _CH, _CH)
                copies.append(
                    pltpu.async_copy(
                        disp_hbm.at[idx_v.at[sl]], vals_v.at[sl], sem))
            for c in copies:
                c.wait()
            return carry

        lax.fori_loop(0, _NCHUNK // _NBUF, gather_group, 0)

        def body(k, acc):
            sl = pl.ds(k * _L, _L)
            v1 = vals_v[sl]
            v2 = vals_v[pl.ds(_K + k * _L, _L)]
            v3 = vals_v[pl.ds(2 * _K + k * _L, _L)]
            lm = lam_v[sl]
            return acc + jnp.abs(lm * (v2 - v1) - (v3 - v1))

        acc = lax.fori_loop(0, _K // _L, body, jnp.zeros((_L,), jnp.float32))
        part_v[...] = acc
        pltpu.sync_copy(part_v, out_hbm.at[w])

    return run(disp_flat, idx2d, lam2d)


def kernel(disp_preds, keysets, lambda_sets):
    gamma = 0.8
    weight = 1.0
    n_preds = disp_preds.shape[0]
    bs = disp_preds.shape[1]
    k = keysets.shape[-1]

    disp_flat = disp_preds.reshape(-1)
    idx2d = keysets.reshape(bs, 3 * k)
    lam2d = lambda_sets.reshape(bs, k)

    parts = _sc_fsloss(disp_flat, idx2d, lam2d)
    psum = parts.sum(axis=1).reshape(n_preds, bs).sum(axis=1)
    weights = gamma ** jnp.arange(n_preds - 1, -1, -1, dtype=jnp.float32)
    return (psum * weights).sum() / (bs * k) * weight


# speedup vs baseline: 3.0026x; 1.1503x over previous
import functools

import jax
import jax.numpy as jnp
from jax import lax
from jax.experimental import pallas as pl
from jax.experimental.pallas import tpu as pltpu
from jax.experimental.pallas import tpu_sc as plsc

_NC = 2
_NS = 16
_NW = _NC * _NS
_L = 16

_HW = 512 * 512
_K = 4096
_NIDX = 3 * _K
_CH = 128
_NCHUNK = _NIDX // _CH
_NBUF = 8


def _sc_fsloss(disp_flat, idx1d, lam1d):
    mesh = plsc.VectorSubcoreMesh(core_axis_name="c", subcore_axis_name="s")

    @functools.partial(
        pl.kernel,
        out_type=jax.ShapeDtypeStruct((_NW * _L,), jnp.float32),
        mesh=mesh,
        scratch_types=[
            pltpu.VMEM((_NIDX,), jnp.int32),
            pltpu.VMEM((_NIDX,), jnp.float32),
            pltpu.VMEM((_K,), jnp.float32),
            pltpu.VMEM((_L,), jnp.float32),
            pltpu.SemaphoreType.DMA,
        ],
    )
    def run(disp_hbm, idx_hbm, lam_hbm, out_hbm, idx_v, vals_v, lam_v,
            part_v, sem):
        w = lax.axis_index("s") * _NC + lax.axis_index("c")
        b = lax.rem(w, 4)

        pltpu.sync_copy(idx_hbm.at[pl.ds(b * _NIDX, _NIDX)], idx_v)
        pltpu.sync_copy(lam_hbm.at[pl.ds(b * _K, _K)], lam_v)

        offv = jnp.full((_L,), w * _HW, dtype=jnp.int32)

        nblk = 8
        blk = _NIDX // nblk

        def gather_block(g, carry):
            def add_off(k, carry2):
                sl = pl.ds(g * blk + k * _L, _L)
                idx_v[sl] = idx_v[sl] + offv
                return carry2

            lax.fori_loop(0, blk // _L, add_off, 0)
            sl = pl.ds(g * blk, blk)
            pltpu.async_copy(disp_hbm.at[idx_v.at[sl]], vals_v.at[sl], sem)
            return carry

        lax.fori_loop(0, nblk, gather_block, 0)

        def drain(g, carry):
            sl = pl.ds(g * blk, blk)
            pltpu.make_async_copy(
                disp_hbm.at[idx_v.at[sl]], vals_v.at[sl], sem).wait()
            return carry

        lax.fori_loop(0, nblk, drain, 0)

        def body(k, acc):
            sl = pl.ds(k * _L, _L)
            v1 = vals_v[sl]
            v2 = vals_v[pl.ds(_K + k * _L, _L)]
            v3 = vals_v[pl.ds(2 * _K + k * _L, _L)]
            lm = lam_v[sl]
            return acc + jnp.abs(lm * (v2 - v1) - (v3 - v1))

        acc = lax.fori_loop(0, _K // _L, body, jnp.zeros((_L,), jnp.float32))
        part_v[...] = acc
        pltpu.sync_copy(part_v, out_hbm.at[pl.ds(w * _L, _L)])

    return run(disp_flat, idx1d, lam1d)


def kernel(disp_preds, keysets, lambda_sets):
    gamma = 0.8
    weight = 1.0
    n_preds = disp_preds.shape[0]
    bs = disp_preds.shape[1]
    k = keysets.shape[-1]

    disp_flat = disp_preds.reshape(-1)
    idx1d = keysets.reshape(-1)
    lam1d = lambda_sets.reshape(-1)

    parts = _sc_fsloss(disp_flat, idx1d, lam1d)
    psum = parts.reshape(n_preds, bs, _L).sum(axis=(1, 2))
    weights = gamma ** jnp.arange(n_preds - 1, -1, -1, dtype=jnp.float32)
    return (psum * weights).sum() / (bs * k) * weight


# speedup vs baseline: 3.0709x; 1.0227x over previous
import functools

import jax
import jax.numpy as jnp
from jax import lax
from jax.experimental import pallas as pl
from jax.experimental.pallas import tpu as pltpu
from jax.experimental.pallas import tpu_sc as plsc

_NC = 2
_NS = 16
_NW = _NC * _NS
_L = 16

_HW = 512 * 512
_K = 4096
_NIDX = 3 * _K
_CH = 128
_NCHUNK = _NIDX // _CH
_NBUF = 8


def _sc_fsloss(disp_flat, idx1d, lam1d):
    mesh = plsc.VectorSubcoreMesh(core_axis_name="c", subcore_axis_name="s")

    @functools.partial(
        pl.kernel,
        out_type=jax.ShapeDtypeStruct((_NW * _L,), jnp.float32),
        mesh=mesh,
        scratch_types=[
            pltpu.VMEM((_NIDX,), jnp.int32),
            pltpu.VMEM((_NIDX,), jnp.float32),
            pltpu.VMEM((_K,), jnp.float32),
            pltpu.VMEM((_L,), jnp.float32),
            pltpu.SemaphoreType.DMA,
            pltpu.SemaphoreType.DMA,
        ],
    )
    def run(disp_hbm, idx_hbm, lam_hbm, out_hbm, idx_v, vals_v, lam_v,
            part_v, sem, lsem):
        w = lax.axis_index("s") * _NC + lax.axis_index("c")
        b = lax.rem(w, 4)

        lam_copy = pltpu.async_copy(lam_hbm.at[pl.ds(b * _K, _K)], lam_v, lsem)
        pltpu.sync_copy(idx_hbm.at[pl.ds(b * _NIDX, _NIDX)], idx_v)

        offv = jnp.full((_L,), w * _HW, dtype=jnp.int32)

        nblk = 8
        blk = _NIDX // nblk

        def gather_block(g, carry):
            def add_off(k, carry2):
                s0 = pl.ds(g * blk + k * 2 * _L, _L)
                s1 = pl.ds(g * blk + k * 2 * _L + _L, _L)
                idx_v[s0] = idx_v[s0] + offv
                idx_v[s1] = idx_v[s1] + offv
                return carry2

            lax.fori_loop(0, blk // (2 * _L), add_off, 0)
            sl = pl.ds(g * blk, blk)
            pltpu.async_copy(disp_hbm.at[idx_v.at[sl]], vals_v.at[sl], sem)
            return carry

        lax.fori_loop(0, nblk, gather_block, 0)
        lam_copy.wait()

        def drain(g, carry):
            sl = pl.ds(g * blk, blk)
            pltpu.make_async_copy(
                disp_hbm.at[idx_v.at[sl]], vals_v.at[sl], sem).wait()
            return carry

        lax.fori_loop(0, nblk, drain, 0)

        def body(k, accs):
            a0, a1 = accs
            s0 = pl.ds(k * 2 * _L, _L)
            s1 = pl.ds(k * 2 * _L + _L, _L)
            v1a = vals_v[s0]
            v2a = vals_v[pl.ds(_K + k * 2 * _L, _L)]
            v3a = vals_v[pl.ds(2 * _K + k * 2 * _L, _L)]
            lma = lam_v[s0]
            v1b = vals_v[s1]
            v2b = vals_v[pl.ds(_K + k * 2 * _L + _L, _L)]
            v3b = vals_v[pl.ds(2 * _K + k * 2 * _L + _L, _L)]
            lmb = lam_v[s1]
            a0 = a0 + jnp.abs(lma * (v2a - v1a) - (v3a - v1a))
            a1 = a1 + jnp.abs(lmb * (v2b - v1b) - (v3b - v1b))
            return (a0, a1)

        zero = jnp.zeros((_L,), jnp.float32)
        a0, a1 = lax.fori_loop(0, _K // (2 * _L), body, (zero, zero))
        acc = a0 + a1
        part_v[...] = acc
        pltpu.sync_copy(part_v, out_hbm.at[pl.ds(w * _L, _L)])

    return run(disp_flat, idx1d, lam1d)


def kernel(disp_preds, keysets, lambda_sets):
    gamma = 0.8
    weight = 1.0
    n_preds = disp_preds.shape[0]
    bs = disp_preds.shape[1]
    k = keysets.shape[-1]

    disp_flat = disp_preds.reshape(-1)
    idx1d = keysets.reshape(-1)
    lam1d = lambda_sets.reshape(-1)

    parts = _sc_fsloss(disp_flat, idx1d, lam1d)
    psum = parts.reshape(n_preds, bs, _L).sum(axis=(1, 2))
    weights = gamma ** jnp.arange(n_preds - 1, -1, -1, dtype=jnp.float32)
    return (psum * weights).sum() / (bs * k) * weight


# speedup vs baseline: 3.0740x; 1.0010x over previous
import functools

import jax
import jax.numpy as jnp
from jax import lax
from jax.experimental import pallas as pl
from jax.experimental.pallas import tpu as pltpu
from jax.experimental.pallas import tpu_sc as plsc

_NC = 2
_NS = 16
_NW = _NC * _NS
_L = 16

_HW = 512 * 512
_K = 4096
_NIDX = 3 * _K
_CH = 128
_NCHUNK = _NIDX // _CH
_NBUF = 8


def _sc_fsloss(disp_flat, idx1d, lam1d):
    mesh = plsc.VectorSubcoreMesh(core_axis_name="c", subcore_axis_name="s")

    @functools.partial(
        pl.kernel,
        out_type=jax.ShapeDtypeStruct((_NW * _L,), jnp.float32),
        mesh=mesh,
        scratch_types=[
            pltpu.VMEM((_NIDX,), jnp.int32),
            pltpu.VMEM((_NIDX,), jnp.float32),
            pltpu.VMEM((_K,), jnp.float32),
            pltpu.VMEM((_L,), jnp.float32),
            pltpu.SemaphoreType.DMA,
            pltpu.SemaphoreType.DMA,
        ],
        compiler_params=pltpu.CompilerParams(skip_device_barrier=True),
    )
    def run(disp_hbm, idx_hbm, lam_hbm, out_hbm, idx_v, vals_v, lam_v,
            part_v, sem, lsem):
        w = lax.axis_index("s") * _NC + lax.axis_index("c")
        b = lax.rem(w, 4)

        lam_copy = pltpu.async_copy(lam_hbm.at[pl.ds(b * _K, _K)], lam_v, lsem)
        pltpu.sync_copy(idx_hbm.at[pl.ds(b * _NIDX, _NIDX)], idx_v)

        offv = jnp.full((_L,), w * _HW, dtype=jnp.int32)

        nblk = 8
        blk = _NIDX // nblk

        def gather_block(g, carry):
            def add_off(k, carry2):
                s0 = pl.ds(g * blk + k * 2 * _L, _L)
                s1 = pl.ds(g * blk + k * 2 * _L + _L, _L)
                idx_v[s0] = idx_v[s0] + offv
                idx_v[s1] = idx_v[s1] + offv
                return carry2

            lax.fori_loop(0, blk // (2 * _L), add_off, 0)
            sl = pl.ds(g * blk, blk)
            pltpu.async_copy(disp_hbm.at[idx_v.at[sl]], vals_v.at[sl], sem)
            return carry

        lax.fori_loop(0, nblk, gather_block, 0)
        lam_copy.wait()

        def drain(g, carry):
            sl = pl.ds(g * blk, blk)
            pltpu.make_async_copy(
                disp_hbm.at[idx_v.at[sl]], vals_v.at[sl], sem).wait()
            return carry

        lax.fori_loop(0, nblk, drain, 0)

        def body(k, accs):
            a0, a1 = accs
            s0 = pl.ds(k * 2 * _L, _L)
            s1 = pl.ds(k * 2 * _L + _L, _L)
            v1a = vals_v[s0]
            v2a = vals_v[pl.ds(_K + k * 2 * _L, _L)]
            v3a = vals_v[pl.ds(2 * _K + k * 2 * _L, _L)]
            lma = lam_v[s0]
            v1b = vals_v[s1]
            v2b = vals_v[pl.ds(_K + k * 2 * _L + _L, _L)]
            v3b = vals_v[pl.ds(2 * _K + k * 2 * _L + _L, _L)]
            lmb = lam_v[s1]
            a0 = a0 + jnp.abs(lma * (v2a - v1a) - (v3a - v1a))
            a1 = a1 + jnp.abs(lmb * (v2b - v1b) - (v3b - v1b))
            return (a0, a1)

        zero = jnp.zeros((_L,), jnp.float32)
        a0, a1 = lax.fori_loop(0, _K // (2 * _L), body, (zero, zero))
        acc = a0 + a1
        part_v[...] = acc
        pltpu.sync_copy(part_v, out_hbm.at[pl.ds(w * _L, _L)])

    return run(disp_flat, idx1d, lam1d)


def kernel(disp_preds, keysets, lambda_sets):
    gamma = 0.8
    weight = 1.0
    n_preds = disp_preds.shape[0]
    bs = disp_preds.shape[1]
    k = keysets.shape[-1]

    disp_flat = disp_preds.reshape(-1)
    idx1d = keysets.reshape(-1)
    lam1d = lambda_sets.reshape(-1)

    parts = _sc_fsloss(disp_flat, idx1d, lam1d)
    psum = parts.reshape(n_preds, bs, _L).sum(axis=(1, 2))
    weights = gamma ** jnp.arange(n_preds - 1, -1, -1, dtype=jnp.float32)
    return (psum * weights).sum() / (bs * k) * weight
